# baseline (speedup 1.0000x reference)
import functools

import jax
import jax.numpy as jnp
from jax.experimental import pallas as pl

SMOOTHING = 0.1
CONFIDENCE = 1.0 - SMOOTHING


def _loss_block_kernel(x_ref, t_ref, out_ref, *, n_rows, n_cols):
    i = pl.program_id(0)

    x = x_ref[...]
    t = t_ref[...]

    m = jnp.max(x, axis=1, keepdims=True)
    s = jnp.sum(jnp.exp(x - m), axis=1, keepdims=True)
    lse = jnp.log(s)
    sum_x = jnp.sum(x, axis=1, keepdims=True)

    col = jax.lax.broadcasted_iota(jnp.int32, x.shape, 1)
    x_t = jnp.sum(jnp.where(col == t, x, 0.0), axis=1, keepdims=True)

    lp_sum = sum_x - n_cols * (m + lse)
    lp_t = x_t - m - lse

    smooth = SMOOTHING / (n_cols - 1)
    loss = -smooth * lp_sum - (CONFIDENCE - smooth) * lp_t

    partial = jnp.sum(loss, axis=(0, 1), keepdims=True) * (1.0 / n_rows)

    @pl.when(i == 0)
    def _():
        out_ref[...] = jnp.zeros_like(out_ref)

    out_ref[...] += partial


def kernel(x, target):
    n_rows, n_cols = x.shape
    block_rows = 1024
    grid = (n_rows // block_rows,)

    t2d = target.astype(jnp.int32).reshape(n_rows, 1)

    out = pl.pallas_call(
        functools.partial(_loss_block_kernel, n_rows=n_rows, n_cols=n_cols),
        grid=grid,
        in_specs=[
            pl.BlockSpec((block_rows, n_cols), lambda i: (i, 0)),
            pl.BlockSpec((block_rows, 1), lambda i: (i, 0)),
        ],
        out_specs=pl.BlockSpec((1, 1), lambda i: (0, 0)),
        out_shape=jax.ShapeDtypeStruct((1, 1), jnp.float32),
    )(x, t2d)
    return out[0, 0]


# speedup vs baseline: 1.3331x; 1.3331x over previous
import functools

import jax
import jax.numpy as jnp
from jax.experimental import pallas as pl

SMOOTHING = 0.1
CONFIDENCE = 1.0 - SMOOTHING


def _loss_block_kernel(x_ref, t_ref, out_ref, *, n_rows, n_cols):
    i = pl.program_id(0)

    x = x_ref[...]
    t = t_ref[...].reshape(-1, 1)

    m = jnp.max(x, axis=1, keepdims=True)
    s = jnp.sum(jnp.exp(x - m), axis=1, keepdims=True)
    lse = jnp.log(s)
    sum_x = jnp.sum(x, axis=1, keepdims=True)

    col = jax.lax.broadcasted_iota(jnp.int32, x.shape, 1)
    x_t = jnp.sum(jnp.where(col == t, x, 0.0), axis=1, keepdims=True)

    lp_sum = sum_x - n_cols * (m + lse)
    lp_t = x_t - m - lse

    smooth = SMOOTHING / (n_cols - 1)
    loss = -smooth * lp_sum - (CONFIDENCE - smooth) * lp_t

    partial = jnp.sum(loss, axis=(0, 1), keepdims=True) * (1.0 / n_rows)

    @pl.when(i == 0)
    def _():
        out_ref[...] = jnp.zeros_like(out_ref)

    out_ref[...] += partial


def kernel(x, target):
    n_rows, n_cols = x.shape
    block_rows = 1024
    grid = (n_rows // block_rows,)

    trow = target.astype(jnp.int32).reshape(1, n_rows)

    out = pl.pallas_call(
        functools.partial(_loss_block_kernel, n_rows=n_rows, n_cols=n_cols),
        grid=grid,
        in_specs=[
            pl.BlockSpec((block_rows, n_cols), lambda i: (i, 0)),
            pl.BlockSpec((1, block_rows), lambda i: (0, i)),
        ],
        out_specs=pl.BlockSpec((1, 1), lambda i: (0, 0)),
        out_shape=jax.ShapeDtypeStruct((1, 1), jnp.float32),
    )(x, trow)
    return out[0, 0]
